# Initial kernel scaffold; baseline (speedup 1.0000x reference)
import jax
import jax.numpy as jnp
from jax.experimental import pallas as pl


def kernel(x, gate_w, w1, w2, w3):
    raise NotImplementedError("write your pallas kernel here")


# speedup vs baseline: 1.1565x; 1.1565x over previous
import functools

import jax
import jax.numpy as jnp
from jax.experimental import pallas as pl
from jax.experimental.pallas import tpu as pltpu

HIDDEN = 1024
FFN = 2048
NUM_EXPERTS = 8
TOP_K = 2

BM = 512
BF = 512


def _moe_body(x_ref, gw_ref, w1_ref, w2_ref, w3_ref,
              out_ref, log_ref, topw_ref, topi_ref):
    e = pl.program_id(1)
    f = pl.program_id(2)
    first = (e == 0) & (f == 0)

    @pl.when(first)
    def _router():
        xb = x_ref[...]
        logits = jnp.dot(xb, gw_ref[...], preferred_element_type=jnp.float32)
        log_ref[...] = logits
        m = jnp.max(logits, axis=-1, keepdims=True)
        p = jnp.exp(logits - m)
        p = p / jnp.sum(p, axis=-1, keepdims=True)
        iota = jax.lax.broadcasted_iota(jnp.int32, p.shape, 1)
        m1 = jnp.max(p, axis=-1, keepdims=True)
        i1 = jnp.min(jnp.where(p == m1, iota, NUM_EXPERTS), axis=-1, keepdims=True)
        p2 = jnp.where(iota == i1, -1.0, p)
        m2 = jnp.max(p2, axis=-1, keepdims=True)
        i2 = jnp.min(jnp.where(p2 == m2, iota, NUM_EXPERTS), axis=-1, keepdims=True)
        s = m1 + m2
        topw_ref[...] = jnp.concatenate([m1 / s, m2 / s], axis=1)
        topi_ref[...] = jnp.concatenate([i1, i2], axis=1)

    xb = x_ref[...]
    h1 = jnp.dot(xb, w1_ref[0], preferred_element_type=jnp.float32)
    h3 = jnp.dot(xb, w3_ref[0], preferred_element_type=jnp.float32)
    g = (h1 * jax.lax.logistic(h1)) * h3
    y = jnp.dot(g, w2_ref[0], preferred_element_type=jnp.float32)
    w_e = jnp.sum(jnp.where(topi_ref[...] == e, topw_ref[...], 0.0), axis=1)
    contrib = w_e[:, None] * y

    @pl.when(first)
    def _init():
        out_ref[...] = contrib

    @pl.when(jnp.logical_not(first))
    def _acc():
        out_ref[...] += contrib


@jax.jit
def kernel(x, gate_w, w1, w2, w3):
    B, S, D = x.shape
    T = B * S
    h = x.reshape(T, D)
    nb = T // BM
    nf = FFN // BF

    grid = (nb, NUM_EXPERTS, nf)
    out, logits = pl.pallas_call(
        _moe_body,
        grid=grid,
        in_specs=[
            pl.BlockSpec((BM, D), lambda i, e, f: (i, 0)),
            pl.BlockSpec((D, NUM_EXPERTS), lambda i, e, f: (0, 0)),
            pl.BlockSpec((1, D, BF), lambda i, e, f: (e, 0, f)),
            pl.BlockSpec((1, BF, D), lambda i, e, f: (e, f, 0)),
            pl.BlockSpec((1, D, BF), lambda i, e, f: (e, 0, f)),
        ],
        out_specs=[
            pl.BlockSpec((BM, D), lambda i, e, f: (i, 0)),
            pl.BlockSpec((BM, NUM_EXPERTS), lambda i, e, f: (i, 0)),
        ],
        out_shape=[
            jax.ShapeDtypeStruct((T, D), jnp.float32),
            jax.ShapeDtypeStruct((T, NUM_EXPERTS), jnp.float32),
        ],
        scratch_shapes=[
            pltpu.VMEM((BM, TOP_K), jnp.float32),
            pltpu.VMEM((BM, TOP_K), jnp.int32),
        ],
        compiler_params=pltpu.CompilerParams(
            dimension_semantics=("arbitrary", "arbitrary", "arbitrary"),
        ),
    )(h, gate_w, w1, w2, w3)
    return out.reshape(B, S, D), logits
